# baseline (speedup 1.0000x reference)
import functools

import jax
import jax.numpy as jnp
from jax import lax
from jax.experimental import pallas as pl
from jax.experimental.pallas import tpu as pltpu
from jax.experimental.pallas import tpu_sc as plsc

NUM_USERS = 25000
NN = 50000
EMB = 64
KL = 3
E = 800000

NC = 2
NS = 16
CH = 128

N_PAD = ((NN + 1 + 127) // 128) * 128
RT = N_PAD // NS
ZR = RT // 23
NSEG = 23

CHUNKS = ((E + CH * NC * NS - 1) // (CH * NC * NS)) * (NC * NS)
E_PAD = CHUNKS * CH
CPT = CHUNKS // NS
CPW = CHUNKS // (NC * NS)
CW = 32
G = 2
GB = 2

_SC_PARAMS = pltpu.CompilerParams(use_tc_tiling_on_sc=False)


def _mesh():
  return plsc.VectorSubcoreMesh(
      core_axis_name="c", subcore_axis_name="s", num_cores=NC, num_subcores=NS
  )


def _bincount_body(srcp, ones_hbm, zhalf_hbm, cnt_out, shared, ones_v,
                   zba, zbb, sidx, semz, semw0, semw1, sems0, sems1):
  c = lax.axis_index("c")
  s = lax.axis_index("s")
  base = s * RT
  zb = (zba, zbb)
  semw = (semw0, semw1)
  sems = (sems0, sems1)

  pltpu.sync_copy(zhalf_hbm, zba)
  zcs = [pltpu.async_copy(zba, shared.at[pl.ds(base + h * ZR, ZR), :], semz)
         for h in range(NSEG)]
  pltpu.sync_copy(ones_hbm, ones_v)
  for d in zcs:
    d.wait()
  plsc.subcore_barrier()

  w = c * NS + s
  cbase = w * CPW
  NGB = CPW // GB

  def idx_fire(b, grp):
    pltpu.async_copy(srcp.at[pl.ds(cbase + grp * GB, GB), :], sidx.at[b],
                     semw[b])

  def idx_wait(b):
    pltpu.make_async_copy(srcp.at[pl.ds(0, GB), :], sidx.at[b],
                          semw[b]).wait()

  def scat_fire(b):
    for q in range(GB):
      pltpu.async_copy(ones_v, shared.at[sidx.at[b, q]], sems[b], add=True)

  def scat_wait(b):
    for q in range(GB):
      pltpu.make_async_copy(ones_v, shared.at[sidx.at[b, q]],
                            sems[b]).wait()

  idx_fire(0, 0)
  idx_wait(0)

  def pair(i, carry):
    @pl.when(i > 0)
    def _():
      scat_wait(1)

    idx_fire(1, 2 * i + 1)
    scat_fire(0)
    idx_wait(1)
    scat_wait(0)
    idx_fire(0, jnp.minimum(2 * i + 2, NGB - 1))
    scat_fire(1)
    idx_wait(0)
    return carry

  lax.fori_loop(0, NGB // 2, pair, 0)
  scat_wait(1)
  plsc.subcore_barrier()

  for h in range(NSEG):
    b = h % 2
    if h >= 2:
      pltpu.make_async_copy(
          zb[b], cnt_out.at[pl.ds(base + (h - 2) * ZR, ZR), c, :],
          semw[b]).wait()
    pltpu.sync_copy(shared.at[pl.ds(base + h * ZR, ZR), :], zb[b])
    pltpu.async_copy(zb[b], cnt_out.at[pl.ds(base + h * ZR, ZR), c, :],
                     semw[b])
  for h in (NSEG - 2, NSEG - 1):
    b = h % 2
    pltpu.make_async_copy(
        zb[b], cnt_out.at[pl.ds(base + h * ZR, ZR), c, :], semw[b]).wait()


@functools.cache
def _bincount_call():
  return pl.kernel(
      _bincount_body,
      out_type=jax.ShapeDtypeStruct((N_PAD, NC, CW), jnp.float32),
      mesh=_mesh(),
      scratch_types=[
          pltpu.VMEM_SHARED((N_PAD, CW), jnp.float32),
          pltpu.VMEM((CH, CW), jnp.float32),
          pltpu.VMEM((ZR, CW), jnp.float32),
          pltpu.VMEM((ZR, CW), jnp.float32),
          pltpu.VMEM((2, GB, CH), jnp.int32),
          pltpu.SemaphoreType.DMA,
          pltpu.SemaphoreType.DMA,
          pltpu.SemaphoreType.DMA,
          pltpu.SemaphoreType.DMA,
          pltpu.SemaphoreType.DMA,
      ],
      compiler_params=_SC_PARAMS,
  )


def _layer_body(gtab, srcp, dstp, zhalf_hbm, agg_out, shared, zba, zbb,
                sidx, didx, sdidx, rows,
                semi0, semi1, semg0, semg1, sems0, sems1, semz):
  c = lax.axis_index("c")
  s = lax.axis_index("s")
  base = s * RT
  coff = c
  zb = (zba, zbb)

  pltpu.sync_copy(zhalf_hbm, zba)
  zcs = [pltpu.async_copy(zba, shared.at[pl.ds(base + h * ZR, ZR), :], semz)
         for h in range(NSEG)]
  for d in zcs:
    d.wait()
  plsc.subcore_barrier()

  cbase = s * CPT
  NG = CPT // G
  semi = (semi0, semi1)
  semg = (semg0, semg1)
  sems = (sems0, sems1)

  def idx_fire(b, grp):
    crow = cbase + grp * G
    pltpu.async_copy(srcp.at[pl.ds(crow, G), :], sidx.at[b], semi[b])
    pltpu.async_copy(dstp.at[pl.ds(crow, G), :], didx.at[b], semi[b])

  def idx_wait(b):
    pltpu.make_async_copy(srcp.at[pl.ds(0, G), :], sidx.at[b], semi[b]).wait()
    pltpu.make_async_copy(dstp.at[pl.ds(0, G), :], didx.at[b], semi[b]).wait()
    for q in range(G):
      for t in range(CH // 16):
        sl = pl.ds(t * 16, 16)
        sidx[b, q, sl] = sidx[b, q, sl] * 2 + coff

  def gath_fire(b):
    for q in range(G):
      pltpu.async_copy(gtab.at[sidx.at[b, q]], rows.at[b, q], semg[b])

  def gath_wait(b):
    for q in range(G):
      pltpu.make_async_copy(gtab.at[sidx.at[b, q]], rows.at[b, q],
                            semg[b]).wait()

  def scat_fire(b):
    for q in range(G):
      for t in range(CH // 16):
        sdidx[b, q, pl.ds(t * 16, 16)] = didx[b, q, pl.ds(t * 16, 16)]
    for q in range(G):
      pltpu.async_copy(rows.at[b, q], shared.at[sdidx.at[b, q]], sems[b],
                       add=True)

  def scat_wait(b):
    for q in range(G):
      pltpu.make_async_copy(rows.at[b, q], shared.at[sdidx.at[b, q]],
                            sems[b]).wait()

  idx_fire(0, 0)
  idx_wait(0)
  gath_fire(0)

  def pair(i, carry):
    g0 = 2 * i
    gath_wait(0)

    @pl.when(i > 0)
    def _():
      scat_wait(1)

    idx_fire(1, g0 + 1)
    scat_fire(0)
    idx_wait(1)
    gath_fire(1)
    g2 = jnp.minimum(g0 + 2, NG - 1)
    idx_fire(0, g2)
    scat_wait(0)
    idx_wait(0)
    gath_fire(0)
    gath_wait(1)
    scat_fire(1)
    return carry

  lax.fori_loop(0, NG // 2, pair, 0)
  gath_wait(0)
  scat_wait(1)
  plsc.subcore_barrier()

  for h in range(NSEG):
    b = h % 2
    if h >= 2:
      pltpu.make_async_copy(
          zb[b], agg_out.at[pl.ds(base + (h - 2) * ZR, ZR), c, :],
          semi[b]).wait()
    pltpu.sync_copy(shared.at[pl.ds(base + h * ZR, ZR), :], zb[b])
    pltpu.async_copy(zb[b], agg_out.at[pl.ds(base + h * ZR, ZR), c, :],
                     semi[b])
  for h in (NSEG - 2, NSEG - 1):
    b = h % 2
    pltpu.make_async_copy(
        zb[b], agg_out.at[pl.ds(base + h * ZR, ZR), c, :], semi[b]).wait()


@functools.cache
def _layer_call():
  return pl.kernel(
      _layer_body,
      out_type=jax.ShapeDtypeStruct((N_PAD, NC, 32), jnp.float32),
      mesh=_mesh(),
      scratch_types=[
          pltpu.VMEM_SHARED((N_PAD, 32), jnp.float32),
          pltpu.VMEM((ZR, 32), jnp.float32),
          pltpu.VMEM((ZR, 32), jnp.float32),
          pltpu.VMEM((2, G, CH), jnp.int32),
          pltpu.VMEM((2, G, CH), jnp.int32),
          pltpu.VMEM((2, G, CH), jnp.int32),
          pltpu.VMEM((2, G, CH, 32), jnp.float32),
          pltpu.SemaphoreType.DMA,
          pltpu.SemaphoreType.DMA,
          pltpu.SemaphoreType.DMA,
          pltpu.SemaphoreType.DMA,
          pltpu.SemaphoreType.DMA,
          pltpu.SemaphoreType.DMA,
          pltpu.SemaphoreType.DMA,
      ],
      compiler_params=_SC_PARAMS,
  )


FR = N_PAD * EMB // 128
BB = 544
NBF = FR // BB


def _nd_body(cnt_ref, nd_ref):
  x = cnt_ref[...]
  a = jnp.maximum(x[:, 0:32] + x[:, 32:64], 1.0)
  b = jnp.maximum(x[:, 64:96] + x[:, 96:128], 1.0)
  na = lax.rsqrt(a)
  nb = lax.rsqrt(b)
  nd_ref[...] = jnp.concatenate([na, na, nb, nb], axis=1)


_nd_call = pl.pallas_call(
    _nd_body,
    grid=(NBF,),
    in_specs=[pl.BlockSpec((BB, 128), lambda t: (t, 0))],
    out_specs=pl.BlockSpec((BB, 128), lambda t: (t, 0)),
    out_shape=jax.ShapeDtypeStruct((FR, 128), jnp.float32),
)


def _init_body(emb_ref, nd_ref, g_ref):
  g_ref[...] = emb_ref[...] * nd_ref[...]


_init_call = pl.pallas_call(
    _init_body,
    grid=(NBF,),
    in_specs=[
        pl.BlockSpec((BB, 128), lambda t: (t, 0)),
        pl.BlockSpec((BB, 128), lambda t: (t, 0)),
    ],
    out_specs=pl.BlockSpec((BB, 128), lambda t: (t, 0)),
    out_shape=jax.ShapeDtypeStruct((FR, 128), jnp.float32),
)


def _scale_mid_body(agg_ref, nd_ref, ps_ref, ns_ref, g_ref):
  e = agg_ref[...] * nd_ref[...]
  ns_ref[...] = ps_ref[...] + e
  g_ref[...] = e * nd_ref[...]


_scale_mid_call = pl.pallas_call(
    _scale_mid_body,
    grid=(NBF,),
    in_specs=[pl.BlockSpec((BB, 128), lambda t: (t, 0))] * 3,
    out_specs=[pl.BlockSpec((BB, 128), lambda t: (t, 0))] * 2,
    out_shape=[
        jax.ShapeDtypeStruct((FR, 128), jnp.float32),
        jax.ShapeDtypeStruct((FR, 128), jnp.float32),
    ],
)


def _scale_last_body(agg_ref, nd_ref, ps_ref, ns_ref):
  e = agg_ref[...] * nd_ref[...]
  ns_ref[...] = (ps_ref[...] + e) * (1.0 / (KL + 1))


_scale_last_call = pl.pallas_call(
    _scale_last_body,
    grid=(NBF,),
    in_specs=[pl.BlockSpec((BB, 128), lambda t: (t, 0))] * 3,
    out_specs=pl.BlockSpec((BB, 128), lambda t: (t, 0)),
    out_shape=jax.ShapeDtypeStruct((FR, 128), jnp.float32),
)


@jax.jit
def kernel(edge_index, users_emb, items_emb):
  src = edge_index[0].astype(jnp.int32)
  dst = edge_index[1].astype(jnp.int32)
  pad = jnp.full((E_PAD - E,), NN, jnp.int32)
  srcp = jnp.concatenate([src, pad]).reshape(CHUNKS, CH)
  dstp = jnp.concatenate([dst, pad]).reshape(CHUNKS, CH)

  emb = jnp.concatenate([users_emb, items_emb], axis=0)
  emb_pad = jnp.pad(emb, ((0, N_PAD - NN), (0, 0)))
  emb2f = emb_pad.reshape(FR, 128)

  ones_rows = jnp.ones((CH, CW), jnp.float32)
  zhalf = jnp.zeros((ZR, 32), jnp.float32)

  cnt = _bincount_call()(srcp, ones_rows, zhalf)
  nd128 = _nd_call(cnt.reshape(FR, 128))

  g = _init_call(emb2f, nd128)
  run_sum = emb2f
  for k in range(KL):
    agg = _layer_call()(g.reshape(NC * N_PAD, 32), srcp, dstp, zhalf)
    aggf = agg.reshape(FR, 128)
    if k == KL - 1:
      run_sum = _scale_last_call(aggf, nd128, run_sum)
    else:
      run_sum, g = _scale_mid_call(aggf, nd128, run_sum)

  out = run_sum.reshape(N_PAD, EMB)[:NN]
  return out[:NUM_USERS], out[NUM_USERS:]


# speedup vs baseline: 1.0199x; 1.0199x over previous
import functools

import jax
import jax.numpy as jnp
from jax import lax
from jax.experimental import pallas as pl
from jax.experimental.pallas import tpu as pltpu
from jax.experimental.pallas import tpu_sc as plsc

NUM_USERS = 25000
NN = 50000
EMB = 64
KL = 3
E = 800000

NC = 2
NS = 16
CH = 128

N_PAD = ((NN + 1 + 127) // 128) * 128
RT = N_PAD // NS
ZR = RT // 23
NSEG = 23

CHUNKS = ((E + CH * NC * NS - 1) // (CH * NC * NS)) * (NC * NS)
E_PAD = CHUNKS * CH
CPT = CHUNKS // NS
CPW = CHUNKS // (NC * NS)
CW = 32
G = 2
GB = 2

_SC_PARAMS = pltpu.CompilerParams(use_tc_tiling_on_sc=False)


def _mesh():
  return plsc.VectorSubcoreMesh(
      core_axis_name="c", subcore_axis_name="s", num_cores=NC, num_subcores=NS
  )


def _bincount_body(srcp, ones_hbm, zhalf_hbm, cnt_out, shared, ones_v,
                   zba, zbb, sidx, semz, semw0, semw1, sems0, sems1):
  c = lax.axis_index("c")
  s = lax.axis_index("s")
  base = s * RT
  zb = (zba, zbb)
  semw = (semw0, semw1)
  sems = (sems0, sems1)

  pltpu.sync_copy(zhalf_hbm, zba)
  zcs = [pltpu.async_copy(zba, shared.at[pl.ds(base + h * ZR, ZR), :], semz)
         for h in range(NSEG)]
  pltpu.sync_copy(ones_hbm, ones_v)
  for d in zcs:
    d.wait()
  plsc.subcore_barrier()

  w = c * NS + s
  cbase = w * CPW
  NGB = CPW // GB

  def idx_fire(b, grp):
    pltpu.async_copy(srcp.at[pl.ds(cbase + grp * GB, GB), :], sidx.at[b],
                     semw[b])

  def idx_wait(b):
    pltpu.make_async_copy(srcp.at[pl.ds(0, GB), :], sidx.at[b],
                          semw[b]).wait()

  def scat_fire(b):
    for q in range(GB):
      pltpu.async_copy(ones_v, shared.at[sidx.at[b, q]], sems[b], add=True)

  def scat_wait(b):
    for q in range(GB):
      pltpu.make_async_copy(ones_v, shared.at[sidx.at[b, q]],
                            sems[b]).wait()

  idx_fire(0, 0)
  idx_wait(0)

  def pair(i, carry):
    @pl.when(i > 0)
    def _():
      scat_wait(1)

    idx_fire(1, 2 * i + 1)
    scat_fire(0)
    idx_wait(1)
    scat_wait(0)
    idx_fire(0, jnp.minimum(2 * i + 2, NGB - 1))
    scat_fire(1)
    idx_wait(0)
    return carry

  lax.fori_loop(0, NGB // 2, pair, 0)
  scat_wait(1)
  plsc.subcore_barrier()

  for h in range(NSEG):
    b = h % 2
    if h >= 2:
      pltpu.make_async_copy(
          zb[b], cnt_out.at[pl.ds(c * N_PAD + base + (h - 2) * ZR, ZR), :],
          semw[b]).wait()
    pltpu.sync_copy(shared.at[pl.ds(base + h * ZR, ZR), :], zb[b])
    pltpu.async_copy(zb[b],
                     cnt_out.at[pl.ds(c * N_PAD + base + h * ZR, ZR), :],
                     semw[b])
  for h in (NSEG - 2, NSEG - 1):
    b = h % 2
    pltpu.make_async_copy(
        zb[b], cnt_out.at[pl.ds(c * N_PAD + base + h * ZR, ZR), :],
        semw[b]).wait()


@functools.cache
def _bincount_call():
  return pl.kernel(
      _bincount_body,
      out_type=jax.ShapeDtypeStruct((NC * N_PAD, CW), jnp.float32),
      mesh=_mesh(),
      scratch_types=[
          pltpu.VMEM_SHARED((N_PAD, CW), jnp.float32),
          pltpu.VMEM((CH, CW), jnp.float32),
          pltpu.VMEM((ZR, CW), jnp.float32),
          pltpu.VMEM((ZR, CW), jnp.float32),
          pltpu.VMEM((2, GB, CH), jnp.int32),
          pltpu.SemaphoreType.DMA,
          pltpu.SemaphoreType.DMA,
          pltpu.SemaphoreType.DMA,
          pltpu.SemaphoreType.DMA,
          pltpu.SemaphoreType.DMA,
      ],
      compiler_params=_SC_PARAMS,
  )


def _layer_body(gtab, src2, dst2, zhalf_hbm, agg_out, shared, zba, zbb,
                sidx, didx, sdidx, rows,
                semi0, semi1, semg0, semg1, sems0, sems1, semz):
  c = lax.axis_index("c")
  s = lax.axis_index("s")
  base = s * RT
  coff = c * N_PAD
  zb = (zba, zbb)

  pltpu.sync_copy(zhalf_hbm, zba)
  zcs = [pltpu.async_copy(zba, shared.at[pl.ds(base + h * ZR, ZR), :], semz)
         for h in range(NSEG)]
  for d in zcs:
    d.wait()
  plsc.subcore_barrier()

  cbase = s * CPT
  NG = CPT // G
  semi = (semi0, semi1)
  semg = (semg0, semg1)
  sems = (sems0, sems1)

  def idx_fire(b, grp):
    crow = cbase + grp * G
    pltpu.async_copy(src2.at[pl.ds(c * CHUNKS + crow, G), :], sidx.at[b],
                     semi[b])
    pltpu.async_copy(dst2.at[pl.ds(crow, G), :], didx.at[b], semi[b])

  def idx_wait(b):
    pltpu.make_async_copy(src2.at[pl.ds(0, G), :], sidx.at[b], semi[b]).wait()
    pltpu.make_async_copy(dst2.at[pl.ds(0, G), :], didx.at[b], semi[b]).wait()

  def gath_fire(b):
    for q in range(G):
      pltpu.async_copy(gtab.at[sidx.at[b, q]], rows.at[b, q], semg[b])

  def gath_wait(b):
    for q in range(G):
      pltpu.make_async_copy(gtab.at[sidx.at[b, q]], rows.at[b, q],
                            semg[b]).wait()

  def scat_fire(b):
    for q in range(G):
      for t in range(CH // 16):
        sdidx[b, q, pl.ds(t * 16, 16)] = didx[b, q, pl.ds(t * 16, 16)]
    for q in range(G):
      pltpu.async_copy(rows.at[b, q], shared.at[sdidx.at[b, q]], sems[b],
                       add=True)

  def scat_wait(b):
    for q in range(G):
      pltpu.make_async_copy(rows.at[b, q], shared.at[sdidx.at[b, q]],
                            sems[b]).wait()

  idx_fire(0, 0)
  idx_wait(0)
  gath_fire(0)

  def pair(i, carry):
    g0 = 2 * i
    gath_wait(0)

    @pl.when(i > 0)
    def _():
      scat_wait(1)

    idx_fire(1, g0 + 1)
    scat_fire(0)
    idx_wait(1)
    gath_fire(1)
    g2 = jnp.minimum(g0 + 2, NG - 1)
    idx_fire(0, g2)
    scat_wait(0)
    idx_wait(0)
    gath_fire(0)
    gath_wait(1)
    scat_fire(1)
    return carry

  lax.fori_loop(0, NG // 2, pair, 0)
  gath_wait(0)
  scat_wait(1)
  plsc.subcore_barrier()

  for h in range(NSEG):
    b = h % 2
    if h >= 2:
      pltpu.make_async_copy(
          zb[b], agg_out.at[pl.ds(coff + base + (h - 2) * ZR, ZR), :],
          semi[b]).wait()
    pltpu.sync_copy(shared.at[pl.ds(base + h * ZR, ZR), :], zb[b])
    pltpu.async_copy(zb[b],
                     agg_out.at[pl.ds(coff + base + h * ZR, ZR), :], semi[b])
  for h in (NSEG - 2, NSEG - 1):
    b = h % 2
    pltpu.make_async_copy(
        zb[b], agg_out.at[pl.ds(coff + base + h * ZR, ZR), :],
        semi[b]).wait()


@functools.cache
def _layer_call():
  return pl.kernel(
      _layer_body,
      out_type=jax.ShapeDtypeStruct((NC * N_PAD, 32), jnp.float32),
      mesh=_mesh(),
      scratch_types=[
          pltpu.VMEM_SHARED((N_PAD, 32), jnp.float32),
          pltpu.VMEM((ZR, 32), jnp.float32),
          pltpu.VMEM((ZR, 32), jnp.float32),
          pltpu.VMEM((2, G, CH), jnp.int32),
          pltpu.VMEM((2, G, CH), jnp.int32),
          pltpu.VMEM((2, G, CH), jnp.int32),
          pltpu.VMEM((2, G, CH, 32), jnp.float32),
          pltpu.SemaphoreType.DMA,
          pltpu.SemaphoreType.DMA,
          pltpu.SemaphoreType.DMA,
          pltpu.SemaphoreType.DMA,
          pltpu.SemaphoreType.DMA,
          pltpu.SemaphoreType.DMA,
          pltpu.SemaphoreType.DMA,
      ],
      compiler_params=_SC_PARAMS,
  )


FR = 2 * N_PAD * 32 // 128
HR = FR // 2
BB = 544
NB = HR // BB


def _nd_body(c0_ref, c1_ref, nd_ref):
  cnt = c0_ref[...] + c1_ref[...]
  nd_ref[...] = lax.rsqrt(jnp.maximum(cnt, 1.0))


_nd_call = pl.pallas_call(
    _nd_body,
    grid=(NB,),
    in_specs=[
        pl.BlockSpec((BB, 128), lambda p: (p, 0)),
        pl.BlockSpec((BB, 128), lambda p: (NB + p, 0)),
    ],
    out_specs=pl.BlockSpec((BB, 128), lambda p: (p, 0)),
    out_shape=jax.ShapeDtypeStruct((HR, 128), jnp.float32),
)


def _init_body(emb_ref, nd_ref, g_ref):
  g_ref[...] = emb_ref[...] * nd_ref[...]


_init_call = pl.pallas_call(
    _init_body,
    grid=(2, NB),
    in_specs=[
        pl.BlockSpec((BB, 128), lambda c, p: (c * NB + p, 0)),
        pl.BlockSpec((BB, 128), lambda c, p: (p, 0)),
    ],
    out_specs=pl.BlockSpec((BB, 128), lambda c, p: (c * NB + p, 0)),
    out_shape=jax.ShapeDtypeStruct((FR, 128), jnp.float32),
)


def _scale_mid_body(agg_ref, nd_ref, ps_ref, ns_ref, g_ref):
  e = agg_ref[...] * nd_ref[...]
  ns_ref[...] = ps_ref[...] + e
  g_ref[...] = e * nd_ref[...]


_scale_mid_call = pl.pallas_call(
    _scale_mid_body,
    grid=(2, NB),
    in_specs=[
        pl.BlockSpec((BB, 128), lambda c, p: (c * NB + p, 0)),
        pl.BlockSpec((BB, 128), lambda c, p: (p, 0)),
        pl.BlockSpec((BB, 128), lambda c, p: (c * NB + p, 0)),
    ],
    out_specs=[
        pl.BlockSpec((BB, 128), lambda c, p: (c * NB + p, 0)),
        pl.BlockSpec((BB, 128), lambda c, p: (c * NB + p, 0)),
    ],
    out_shape=[
        jax.ShapeDtypeStruct((FR, 128), jnp.float32),
        jax.ShapeDtypeStruct((FR, 128), jnp.float32),
    ],
)


def _scale_last_body(agg_ref, nd_ref, ps_ref, ns_ref):
  e = agg_ref[...] * nd_ref[...]
  ns_ref[...] = (ps_ref[...] + e) * (1.0 / (KL + 1))


_scale_last_call = pl.pallas_call(
    _scale_last_body,
    grid=(2, NB),
    in_specs=[
        pl.BlockSpec((BB, 128), lambda c, p: (c * NB + p, 0)),
        pl.BlockSpec((BB, 128), lambda c, p: (p, 0)),
        pl.BlockSpec((BB, 128), lambda c, p: (c * NB + p, 0)),
    ],
    out_specs=pl.BlockSpec((BB, 128), lambda c, p: (c * NB + p, 0)),
    out_shape=jax.ShapeDtypeStruct((FR, 128), jnp.float32),
)


@jax.jit
def kernel(edge_index, users_emb, items_emb):
  src = edge_index[0].astype(jnp.int32)
  dst = edge_index[1].astype(jnp.int32)
  pad = jnp.full((E_PAD - E,), NN, jnp.int32)
  srcp = jnp.concatenate([src, pad]).reshape(CHUNKS, CH)
  dstp = jnp.concatenate([dst, pad]).reshape(CHUNKS, CH)
  src2 = jnp.concatenate([srcp, srcp + N_PAD])

  emb = jnp.concatenate([users_emb, items_emb], axis=0)
  emb_pad = jnp.pad(emb, ((0, N_PAD - NN), (0, 0)))
  emb2f = jnp.stack([emb_pad[:, :32], emb_pad[:, 32:]]).reshape(FR, 128)

  ones_rows = jnp.ones((CH, CW), jnp.float32)
  zhalf = jnp.zeros((ZR, 32), jnp.float32)

  cnt = _bincount_call()(srcp, ones_rows, zhalf)
  nd128 = _nd_call(cnt.reshape(FR, 128), cnt.reshape(FR, 128))

  g = _init_call(emb2f, nd128)
  run_sum = emb2f
  for k in range(KL):
    agg = _layer_call()(g.reshape(NC * N_PAD, 32), src2, dstp, zhalf)
    aggf = agg.reshape(FR, 128)
    if k == KL - 1:
      run_sum = _scale_last_call(aggf, nd128, run_sum)
    else:
      run_sum, g = _scale_mid_call(aggf, nd128, run_sum)
      g = g.reshape(FR, 128)

  s2 = run_sum.reshape(2, N_PAD, 32)
  out = jnp.concatenate([s2[0], s2[1]], axis=1)[:NN]
  return out[:NUM_USERS], out[NUM_USERS:]
